# baseline (speedup 1.0000x reference)
import functools

import jax
import jax.numpy as jnp
from jax import lax
from jax.experimental import pallas as pl
from jax.experimental.pallas import tpu as pltpu
from jax.experimental.pallas import tpu_sc as plsc

NC, NS, NW = 2, 16, 32
N_NODES = 10000
N_EDGES = 320000
N_GRAPHS = 64


def _scatter_rows(n_in, n_out, d, c_chunks, k, gather):
    rpt = n_out // NS
    zr = min(128, rpt)
    nz = rpt // zr
    assert n_out % NS == 0 and rpt % zr == 0 and k <= 128
    assert (d % 16 == 0) or not gather
    assert rpt % 8 == 0 and zr % 8 == 0
    assert c_chunks % 2 == 0 or not gather

    nbuf = 4
    mesh = plsc.VectorSubcoreMesh(core_axis_name="c", subcore_axis_name="s")
    scratch = [
        pltpu.VMEM((c_chunks, k), jnp.int32),
        pltpu.VMEM((nbuf, k, d), jnp.float32),
        pltpu.VMEM((zr, d), jnp.float32),
        pltpu.VMEM_SHARED((n_out, d), jnp.float32),
        [pltpu.SemaphoreType.DMA] * nbuf,
        [pltpu.SemaphoreType.DMA] * nbuf,
    ]
    if gather:
        scratch.append(pltpu.VMEM((c_chunks, k), jnp.int32))

    def body(t_hbm, src_hbm, dst_hbm, out_hbm, dst_v, rows_v, zrow_v, out_sh,
             gsems, ssems, src_v=None):
        ci = lax.axis_index("c")
        si = lax.axis_index("s")
        wid = si * NC + ci

        pltpu.sync_copy(dst_hbm.at[wid], dst_v)
        if gather:
            pltpu.sync_copy(src_hbm.at[wid], src_v)
            for b in range(nbuf):
                pltpu.async_copy(t_hbm.at[src_v.at[b]], rows_v.at[b],
                                 gsems[b])

        if gather:
            z16 = jnp.zeros((16,), jnp.float32)

            def zfill(i, carry):
                for tcol in range(d // 16):
                    zrow_v[i, pl.ds(tcol * 16, 16)] = z16
                return carry

            lax.fori_loop(0, zr, zfill, 0)
        else:
            pltpu.sync_copy(t_hbm.at[pl.ds(128, zr)], zrow_v)
        for r in range(nz):
            pltpu.sync_copy(zrow_v, out_sh.at[pl.ds(si * rpt + r * zr, zr)])
        plsc.subcore_barrier()

        if gather:
            assert c_chunks % nbuf == 0

            def gfire(j, b):
                pltpu.async_copy(t_hbm.at[src_v.at[j]], rows_v.at[b],
                                 gsems[b])

            def gwait(j, b):
                pltpu.make_async_copy(t_hbm.at[src_v.at[j]], rows_v.at[b],
                                      gsems[b]).wait()

            def step(i, carry):
                sdescs = []
                for b in range(nbuf):
                    j = i * nbuf + b
                    gwait(j, b)
                    sdescs.append(pltpu.async_copy(
                        rows_v.at[b], out_sh.at[dst_v.at[j]], ssems[b],
                        add=True))
                for b in range(nbuf):
                    sdescs[b].wait()
                    jn = i * nbuf + nbuf + b

                    @pl.when(jn < c_chunks)
                    def _():
                        gfire(jn, b)
                return carry

            lax.fori_loop(0, c_chunks // nbuf, step, 0)
        else:
            pltpu.sync_copy(t_hbm.at[pl.ds(0, k)], rows_v.at[0])

            def step(j, carry):
                pltpu.sync_copy(rows_v.at[0], out_sh.at[dst_v.at[j]], add=True)
                return carry

            lax.fori_loop(0, c_chunks, step, 0)

        del gsems, ssems

        plsc.subcore_barrier()
        for r in range(nz):
            off = si * rpt + r * zr
            pltpu.sync_copy(out_sh.at[pl.ds(off, zr)],
                            out_hbm.at[ci, pl.ds(off, zr)])

    if gather:
        def entry(t_hbm, src_hbm, dst_hbm, out_hbm, dst_v, rows_v, zrow_v,
                  out_sh, gsems, ssems, src_v):
            body(t_hbm, src_hbm, dst_hbm, out_hbm, dst_v, rows_v, zrow_v,
                 out_sh, gsems, ssems, src_v)
    else:
        def entry(t_hbm, src_hbm, dst_hbm, out_hbm, dst_v, rows_v, zrow_v,
                  out_sh, gsems, ssems):
            body(t_hbm, src_hbm, dst_hbm, out_hbm, dst_v, rows_v, zrow_v,
                 out_sh, gsems, ssems)

    return pl.kernel(
        entry,
        out_type=jax.ShapeDtypeStruct((NC, n_out, d), jnp.float32),
        mesh=mesh,
        scratch_types=scratch,
        compiler_params=pltpu.CompilerParams(use_tc_tiling_on_sc=False),
    )


N_PAD = 10240

_deg_call = _scatter_rows(256, N_PAD, 8, 80, 125, gather=False)
_layer64_call = _scatter_rows(N_NODES, N_PAD, 64, 80, 125, gather=True)
_pool_call = _scatter_rows(N_NODES, 128, 128, 4, 80, gather=True)


def _pad2(w, rows, cols):
    return jnp.zeros((rows, cols), w.dtype).at[:w.shape[0], :w.shape[1]].set(w)


def kernel(x, edge_index, batch, W1, b1, W2, b2, W3, b3, W4, b4,
           fcW1, fcb1, fcW2, fcb2):
    src3 = edge_index[0].reshape(NW, 80, 125)
    dst3 = edge_index[1].reshape(NW, 80, 125)

    ones_rows = jnp.zeros((256, 8), jnp.float32).at[:125].set(1.0)
    degp = _deg_call(ones_rows, src3, dst3)
    deg = degp[0, :N_NODES, 0] + degp[1, :N_NODES, 0] + 1.0
    dinv = lax.rsqrt(deg)

    def conv(h, W, b, dpad):
        t = (h @ _pad2(W, h.shape[1], dpad)) * dinv[:, None]
        s_parts = []
        for i in range(dpad // 64):
            tc = t[:, i * 64:(i + 1) * 64]
            p = _layer64_call(tc, src3, dst3)
            s_parts.append(p[0, :N_NODES] + p[1, :N_NODES] + tc)
        s = jnp.concatenate(s_parts, 1) if len(s_parts) > 1 else s_parts[0]
        bp = jnp.zeros((dpad,), b.dtype).at[:b.shape[0]].set(b)
        return jax.nn.relu(s * dinv[:, None] + bp)

    h = conv(x, W1, b1, 64)
    h = conv(h, W2, b2, 64)
    h = conv(h, W3, b3, 64)
    u = h * dinv[:, None]
    p = _layer64_call(u, src3, dst3)
    a4 = (p[0, :N_NODES] + p[1, :N_NODES] + u) * dinv[:, None]
    b4p = jnp.zeros((128,), b4.dtype).at[:b4.shape[0]].set(b4)
    h = jax.nn.relu(a4 @ _pad2(W4, 64, 128) + b4p)

    h = h.at[:, 100].set(1.0)
    npad = 240
    srcp = jnp.concatenate(
        [jnp.arange(N_NODES, dtype=jnp.int32),
         jnp.zeros((npad,), jnp.int32)]).reshape(NW, 4, 80)
    dstp = jnp.concatenate(
        [batch.astype(jnp.int32),
         jnp.full((npad,), N_GRAPHS, jnp.int32)]).reshape(NW, 4, 80)
    pparts = _pool_call(h, srcp, dstp)
    sums = (pparts[0] + pparts[1])[:N_GRAPHS]
    counts = sums[:, 100]
    pooled = sums[:, :100] / jnp.maximum(counts, 1.0)[:, None]

    z = pooled @ fcW1 + fcb1
    z = z @ fcW2 + fcb2
    return z


# speedup vs baseline: 1.0102x; 1.0102x over previous
import functools

import jax
import jax.numpy as jnp
from jax import lax
from jax.experimental import pallas as pl
from jax.experimental.pallas import tpu as pltpu
from jax.experimental.pallas import tpu_sc as plsc

NC, NS, NW = 2, 16, 32
N_NODES = 10000
N_EDGES = 320000
N_GRAPHS = 64


def _scatter_rows(n_in, n_out, d, c_chunks, k, gather):
    rpt = n_out // NS
    zr = min(128, rpt)
    nz = rpt // zr
    assert n_out % NS == 0 and rpt % zr == 0 and k <= 128
    assert (d % 16 == 0) or not gather
    assert rpt % 8 == 0 and zr % 8 == 0
    assert c_chunks % 2 == 0 or not gather

    nbuf = 4
    mesh = plsc.VectorSubcoreMesh(core_axis_name="c", subcore_axis_name="s")
    scratch = [
        pltpu.VMEM((c_chunks, k), jnp.int32),
        pltpu.VMEM((nbuf, k, d), jnp.float32),
        pltpu.VMEM((zr, d), jnp.float32),
        pltpu.VMEM_SHARED((n_out, d), jnp.float32),
        [pltpu.SemaphoreType.DMA] * nbuf,
        [pltpu.SemaphoreType.DMA] * nbuf,
    ]
    if gather:
        scratch.append(pltpu.VMEM((c_chunks, k), jnp.int32))

    def body(t_hbm, src_hbm, dst_hbm, out_hbm, dst_v, rows_v, zrow_v, out_sh,
             gsems, ssems, src_v=None):
        ci = lax.axis_index("c")
        si = lax.axis_index("s")
        wid = si * NC + ci

        pltpu.sync_copy(dst_hbm.at[wid], dst_v)
        if gather:
            pltpu.sync_copy(src_hbm.at[wid], src_v)
            for b in range(nbuf):
                pltpu.async_copy(t_hbm.at[src_v.at[b]], rows_v.at[b],
                                 gsems[b])

        if gather:
            z16 = jnp.zeros((16,), jnp.float32)

            def zfill(i, carry):
                for tcol in range(d // 16):
                    zrow_v[i, pl.ds(tcol * 16, 16)] = z16
                return carry

            lax.fori_loop(0, zr, zfill, 0)
        else:
            pltpu.sync_copy(t_hbm.at[pl.ds(128, zr)], zrow_v)
        for r in range(nz):
            pltpu.sync_copy(zrow_v, out_sh.at[pl.ds(si * rpt + r * zr, zr)])
        plsc.subcore_barrier()

        if gather:
            assert c_chunks % nbuf == 0

            def gfire(j, b):
                pltpu.async_copy(t_hbm.at[src_v.at[j]], rows_v.at[b],
                                 gsems[b])

            def gwait(j, b):
                pltpu.make_async_copy(t_hbm.at[src_v.at[j]], rows_v.at[b],
                                      gsems[b]).wait()

            def step(i, carry):
                sdescs = []
                for b in range(nbuf):
                    j = i * nbuf + b
                    gwait(j, b)
                    sdescs.append(pltpu.async_copy(
                        rows_v.at[b], out_sh.at[dst_v.at[j]], ssems[b],
                        add=True))
                for b in range(nbuf):
                    sdescs[b].wait()
                    jn = i * nbuf + nbuf + b

                    @pl.when(jn < c_chunks)
                    def _():
                        gfire(jn, b)
                return carry

            lax.fori_loop(0, c_chunks // nbuf, step, 0)
        else:
            assert c_chunks % nbuf == 0
            pltpu.sync_copy(t_hbm.at[pl.ds(0, k)], rows_v.at[0])

            def step(i, carry):
                sdescs = [
                    pltpu.async_copy(rows_v.at[0],
                                     out_sh.at[dst_v.at[i * nbuf + b]],
                                     ssems[b], add=True)
                    for b in range(nbuf)
                ]
                for sd in sdescs:
                    sd.wait()
                return carry

            lax.fori_loop(0, c_chunks // nbuf, step, 0)

        plsc.subcore_barrier()
        for r in range(nz):
            off = si * rpt + r * zr
            pltpu.sync_copy(out_sh.at[pl.ds(off, zr)],
                            out_hbm.at[ci, pl.ds(off, zr)])

    if gather:
        def entry(t_hbm, src_hbm, dst_hbm, out_hbm, dst_v, rows_v, zrow_v,
                  out_sh, gsems, ssems, src_v):
            body(t_hbm, src_hbm, dst_hbm, out_hbm, dst_v, rows_v, zrow_v,
                 out_sh, gsems, ssems, src_v)
    else:
        def entry(t_hbm, src_hbm, dst_hbm, out_hbm, dst_v, rows_v, zrow_v,
                  out_sh, gsems, ssems):
            body(t_hbm, src_hbm, dst_hbm, out_hbm, dst_v, rows_v, zrow_v,
                 out_sh, gsems, ssems)

    return pl.kernel(
        entry,
        out_type=jax.ShapeDtypeStruct((NC, n_out, d), jnp.float32),
        mesh=mesh,
        scratch_types=scratch,
        compiler_params=pltpu.CompilerParams(use_tc_tiling_on_sc=False),
    )


N_PAD = 10240

_deg_call = _scatter_rows(256, N_PAD, 8, 80, 125, gather=False)
_layer64_call = _scatter_rows(N_NODES, N_PAD, 64, 80, 125, gather=True)
_pool_call = _scatter_rows(N_NODES, 128, 128, 4, 80, gather=True)


def _pad2(w, rows, cols):
    return jnp.zeros((rows, cols), w.dtype).at[:w.shape[0], :w.shape[1]].set(w)


def kernel(x, edge_index, batch, W1, b1, W2, b2, W3, b3, W4, b4,
           fcW1, fcb1, fcW2, fcb2):
    src3 = edge_index[0].reshape(NW, 80, 125)
    dst3 = edge_index[1].reshape(NW, 80, 125)

    ones_rows = jnp.zeros((256, 8), jnp.float32).at[:125].set(1.0)
    degp = _deg_call(ones_rows, src3, dst3)
    deg = degp[0, :N_NODES, 0] + degp[1, :N_NODES, 0] + 1.0
    dinv = lax.rsqrt(deg)

    def conv(h, W, b, dpad):
        t = (h @ _pad2(W, h.shape[1], dpad)) * dinv[:, None]
        s_parts = []
        for i in range(dpad // 64):
            tc = t[:, i * 64:(i + 1) * 64]
            p = _layer64_call(tc, src3, dst3)
            s_parts.append(p[0, :N_NODES] + p[1, :N_NODES] + tc)
        s = jnp.concatenate(s_parts, 1) if len(s_parts) > 1 else s_parts[0]
        bp = jnp.zeros((dpad,), b.dtype).at[:b.shape[0]].set(b)
        return jax.nn.relu(s * dinv[:, None] + bp)

    h = conv(x, W1, b1, 64)
    h = conv(h, W2, b2, 64)
    h = conv(h, W3, b3, 64)
    u = h * dinv[:, None]
    p = _layer64_call(u, src3, dst3)
    a4 = (p[0, :N_NODES] + p[1, :N_NODES] + u) * dinv[:, None]
    b4p = jnp.zeros((128,), b4.dtype).at[:b4.shape[0]].set(b4)
    h = jax.nn.relu(a4 @ _pad2(W4, 64, 128) + b4p)

    h = h.at[:, 100].set(1.0)
    npad = 240
    srcp = jnp.concatenate(
        [jnp.arange(N_NODES, dtype=jnp.int32),
         jnp.zeros((npad,), jnp.int32)]).reshape(NW, 4, 80)
    dstp = jnp.concatenate(
        [batch.astype(jnp.int32),
         jnp.full((npad,), N_GRAPHS, jnp.int32)]).reshape(NW, 4, 80)
    pparts = _pool_call(h, srcp, dstp)
    sums = (pparts[0] + pparts[1])[:N_GRAPHS]
    counts = sums[:, 100]
    pooled = sums[:, :100] / jnp.maximum(counts, 1.0)[:, None]

    z = pooled @ fcW1 + fcb1
    z = z @ fcW2 + fcb2
    return z


# speedup vs baseline: 1.0208x; 1.0105x over previous
import functools

import jax
import jax.numpy as jnp
from jax import lax
from jax.experimental import pallas as pl
from jax.experimental.pallas import tpu as pltpu
from jax.experimental.pallas import tpu_sc as plsc

NC, NS, NW = 2, 16, 32
N_NODES = 10000
N_EDGES = 320000
N_GRAPHS = 64


def _scatter_rows(n_in, n_out, d, c_chunks, k, gather):
    rpt = n_out // NS
    zr = min(128, rpt)
    nz = rpt // zr
    assert n_out % NS == 0 and rpt % zr == 0 and k <= 128
    assert (d % 16 == 0) or not gather
    assert rpt % 8 == 0 and zr % 8 == 0
    assert c_chunks % 2 == 0 or not gather

    nbuf = 5 if c_chunks % 5 == 0 else 4
    mesh = plsc.VectorSubcoreMesh(core_axis_name="c", subcore_axis_name="s")
    scratch = [
        pltpu.VMEM((c_chunks, k), jnp.int32),
        pltpu.VMEM((nbuf, k, d), jnp.float32),
        pltpu.VMEM((zr, d), jnp.float32),
        pltpu.VMEM_SHARED((n_out, d), jnp.float32),
        [pltpu.SemaphoreType.DMA] * nbuf,
        [pltpu.SemaphoreType.DMA] * nbuf,
    ]
    if gather:
        scratch.append(pltpu.VMEM((c_chunks, k), jnp.int32))

    def body(t_hbm, src_hbm, dst_hbm, out_hbm, dst_v, rows_v, zrow_v, out_sh,
             gsems, ssems, src_v=None):
        ci = lax.axis_index("c")
        si = lax.axis_index("s")
        wid = si * NC + ci

        pltpu.sync_copy(dst_hbm.at[wid], dst_v)
        if gather:
            pltpu.sync_copy(src_hbm.at[wid], src_v)
            for b in range(nbuf):
                pltpu.async_copy(t_hbm.at[src_v.at[b]], rows_v.at[b],
                                 gsems[b])

        if gather:
            z16 = jnp.zeros((16,), jnp.float32)

            def zfill(i, carry):
                for tcol in range(d // 16):
                    zrow_v[i, pl.ds(tcol * 16, 16)] = z16
                return carry

            lax.fori_loop(0, zr, zfill, 0)
        else:
            pltpu.sync_copy(t_hbm.at[pl.ds(128, zr)], zrow_v)
        for r in range(nz):
            pltpu.sync_copy(zrow_v, out_sh.at[pl.ds(si * rpt + r * zr, zr)])
        plsc.subcore_barrier()

        if gather:
            assert c_chunks % nbuf == 0

            def gfire(j, b):
                pltpu.async_copy(t_hbm.at[src_v.at[j]], rows_v.at[b],
                                 gsems[b])

            def gwait(j, b):
                pltpu.make_async_copy(t_hbm.at[src_v.at[j]], rows_v.at[b],
                                      gsems[b]).wait()

            def step(i, carry):
                sdescs = []
                for b in range(nbuf):
                    j = i * nbuf + b
                    gwait(j, b)
                    sdescs.append(pltpu.async_copy(
                        rows_v.at[b], out_sh.at[dst_v.at[j]], ssems[b],
                        add=True))
                for b in range(nbuf):
                    sdescs[b].wait()
                    jn = i * nbuf + nbuf + b

                    @pl.when(jn < c_chunks)
                    def _():
                        gfire(jn, b)
                return carry

            lax.fori_loop(0, c_chunks // nbuf, step, 0)
        else:
            assert c_chunks % nbuf == 0
            pltpu.sync_copy(t_hbm.at[pl.ds(0, k)], rows_v.at[0])

            def step(i, carry):
                sdescs = [
                    pltpu.async_copy(rows_v.at[0],
                                     out_sh.at[dst_v.at[i * nbuf + b]],
                                     ssems[b], add=True)
                    for b in range(nbuf)
                ]
                for sd in sdescs:
                    sd.wait()
                return carry

            lax.fori_loop(0, c_chunks // nbuf, step, 0)

        plsc.subcore_barrier()
        for r in range(nz):
            off = si * rpt + r * zr
            pltpu.sync_copy(out_sh.at[pl.ds(off, zr)],
                            out_hbm.at[ci, pl.ds(off, zr)])

    if gather:
        def entry(t_hbm, src_hbm, dst_hbm, out_hbm, dst_v, rows_v, zrow_v,
                  out_sh, gsems, ssems, src_v):
            body(t_hbm, src_hbm, dst_hbm, out_hbm, dst_v, rows_v, zrow_v,
                 out_sh, gsems, ssems, src_v)
    else:
        def entry(t_hbm, src_hbm, dst_hbm, out_hbm, dst_v, rows_v, zrow_v,
                  out_sh, gsems, ssems):
            body(t_hbm, src_hbm, dst_hbm, out_hbm, dst_v, rows_v, zrow_v,
                 out_sh, gsems, ssems)

    return pl.kernel(
        entry,
        out_type=jax.ShapeDtypeStruct((NC, n_out, d), jnp.float32),
        mesh=mesh,
        scratch_types=scratch,
        compiler_params=pltpu.CompilerParams(use_tc_tiling_on_sc=False),
    )


N_PAD = 10240

_deg_call = _scatter_rows(256, N_PAD, 8, 80, 125, gather=False)
_layer64_call = _scatter_rows(N_NODES, N_PAD, 64, 80, 125, gather=True)
_pool_call = _scatter_rows(N_NODES, 128, 128, 4, 80, gather=True)


def _pad2(w, rows, cols):
    return jnp.zeros((rows, cols), w.dtype).at[:w.shape[0], :w.shape[1]].set(w)


def kernel(x, edge_index, batch, W1, b1, W2, b2, W3, b3, W4, b4,
           fcW1, fcb1, fcW2, fcb2):
    src3 = edge_index[0].reshape(NW, 80, 125)
    dst3 = edge_index[1].reshape(NW, 80, 125)

    ones_rows = jnp.zeros((256, 8), jnp.float32).at[:125].set(1.0)
    degp = _deg_call(ones_rows, src3, dst3)
    deg = degp[0, :N_NODES, 0] + degp[1, :N_NODES, 0] + 1.0
    dinv = lax.rsqrt(deg)

    def conv(h, W, b, dpad):
        t = (h @ _pad2(W, h.shape[1], dpad)) * dinv[:, None]
        s_parts = []
        for i in range(dpad // 64):
            tc = t[:, i * 64:(i + 1) * 64]
            p = _layer64_call(tc, src3, dst3)
            s_parts.append(p[0, :N_NODES] + p[1, :N_NODES] + tc)
        s = jnp.concatenate(s_parts, 1) if len(s_parts) > 1 else s_parts[0]
        bp = jnp.zeros((dpad,), b.dtype).at[:b.shape[0]].set(b)
        return jax.nn.relu(s * dinv[:, None] + bp)

    h = conv(x, W1, b1, 64)
    h = conv(h, W2, b2, 64)
    h = conv(h, W3, b3, 64)
    u = h * dinv[:, None]
    p = _layer64_call(u, src3, dst3)
    a4 = (p[0, :N_NODES] + p[1, :N_NODES] + u) * dinv[:, None]
    b4p = jnp.zeros((128,), b4.dtype).at[:b4.shape[0]].set(b4)
    h = jax.nn.relu(a4 @ _pad2(W4, 64, 128) + b4p)

    h = h.at[:, 100].set(1.0)
    npad = 240
    srcp = jnp.concatenate(
        [jnp.arange(N_NODES, dtype=jnp.int32),
         jnp.zeros((npad,), jnp.int32)]).reshape(NW, 4, 80)
    dstp = jnp.concatenate(
        [batch.astype(jnp.int32),
         jnp.full((npad,), N_GRAPHS, jnp.int32)]).reshape(NW, 4, 80)
    pparts = _pool_call(h, srcp, dstp)
    sums = (pparts[0] + pparts[1])[:N_GRAPHS]
    counts = sums[:, 100]
    pooled = sums[:, :100] / jnp.maximum(counts, 1.0)[:, None]

    z = pooled @ fcW1 + fcb1
    z = z @ fcW2 + fcb2
    return z


# speedup vs baseline: 1.0216x; 1.0008x over previous
import jax
import jax.numpy as jnp
from jax import lax
from jax.experimental import pallas as pl
from jax.experimental.pallas import tpu as pltpu
from jax.experimental.pallas import tpu_sc as plsc

NC, NS, NW = 2, 16, 32
N_NODES = 10000
N_EDGES = 320000
N_GRAPHS = 64


def _scatter_rows(n_in, n_out, d, c_chunks, k, gather):
    rpt = n_out // NS
    zr = min(128, rpt)
    nz = rpt // zr
    assert n_out % NS == 0 and rpt % zr == 0 and k <= 128
    assert (d % 16 == 0) or not gather
    assert rpt % 8 == 0 and zr % 8 == 0
    assert c_chunks % 2 == 0 or not gather

    nbuf = 5 if c_chunks % 5 == 0 else 4
    mesh = plsc.VectorSubcoreMesh(core_axis_name="c", subcore_axis_name="s")
    scratch = [
        pltpu.VMEM((c_chunks, k), jnp.int32),
        pltpu.VMEM((nbuf, k, d), jnp.float32),
        pltpu.VMEM((zr, d), jnp.float32),
        pltpu.VMEM_SHARED((n_out, d), jnp.float32),
        [pltpu.SemaphoreType.DMA] * nbuf,
        [pltpu.SemaphoreType.DMA] * nbuf,
    ]
    if gather:
        scratch.append(pltpu.VMEM((c_chunks, k), jnp.int32))

    def body(t_hbm, src_hbm, dst_hbm, out_hbm, dst_v, rows_v, zrow_v, out_sh,
             gsems, ssems, src_v=None):
        ci = lax.axis_index("c")
        si = lax.axis_index("s")
        wid = si * NC + ci

        pltpu.sync_copy(dst_hbm.at[wid], dst_v)
        if gather:
            pltpu.sync_copy(src_hbm.at[wid], src_v)
            for b in range(nbuf):
                pltpu.async_copy(t_hbm.at[src_v.at[b]], rows_v.at[b],
                                 gsems[b])

        if gather:
            z16 = jnp.zeros((16,), jnp.float32)

            def zfill(i, carry):
                for tcol in range(d // 16):
                    zrow_v[i, pl.ds(tcol * 16, 16)] = z16
                return carry

            lax.fori_loop(0, zr, zfill, 0)
        else:
            pltpu.sync_copy(t_hbm.at[pl.ds(128, zr)], zrow_v)
        for r in range(nz):
            pltpu.sync_copy(zrow_v, out_sh.at[pl.ds(si * rpt + r * zr, zr)])
        plsc.subcore_barrier()

        if gather:
            assert c_chunks % nbuf == 0

            def gfire(j, b):
                pltpu.async_copy(t_hbm.at[src_v.at[j]], rows_v.at[b],
                                 gsems[b])

            def gwait(j, b):
                pltpu.make_async_copy(t_hbm.at[src_v.at[j]], rows_v.at[b],
                                      gsems[b]).wait()

            def step(i, carry):
                sdescs = []
                for b in range(nbuf):
                    j = i * nbuf + b
                    gwait(j, b)
                    sdescs.append(pltpu.async_copy(
                        rows_v.at[b], out_sh.at[dst_v.at[j]], ssems[b],
                        add=True))
                for b in range(nbuf):
                    sdescs[b].wait()
                    jn = i * nbuf + nbuf + b

                    @pl.when(jn < c_chunks)
                    def _():
                        gfire(jn, b)
                return carry

            lax.fori_loop(0, c_chunks // nbuf, step, 0)
        else:
            assert c_chunks % nbuf == 0
            pltpu.sync_copy(t_hbm.at[pl.ds(0, k)], rows_v.at[0])

            def step(i, carry):
                sdescs = [
                    pltpu.async_copy(rows_v.at[0],
                                     out_sh.at[dst_v.at[i * nbuf + b]],
                                     ssems[b], add=True)
                    for b in range(nbuf)
                ]
                for sd in sdescs:
                    sd.wait()
                return carry

            lax.fori_loop(0, c_chunks // nbuf, step, 0)

        plsc.subcore_barrier()
        for r in range(nz):
            off = si * rpt + r * zr
            pltpu.sync_copy(out_sh.at[pl.ds(off, zr)],
                            out_hbm.at[ci, pl.ds(off, zr)])

    if gather:
        def entry(t_hbm, src_hbm, dst_hbm, out_hbm, dst_v, rows_v, zrow_v,
                  out_sh, gsems, ssems, src_v):
            body(t_hbm, src_hbm, dst_hbm, out_hbm, dst_v, rows_v, zrow_v,
                 out_sh, gsems, ssems, src_v)
    else:
        def entry(t_hbm, src_hbm, dst_hbm, out_hbm, dst_v, rows_v, zrow_v,
                  out_sh, gsems, ssems):
            body(t_hbm, src_hbm, dst_hbm, out_hbm, dst_v, rows_v, zrow_v,
                 out_sh, gsems, ssems)

    return pl.kernel(
        entry,
        out_type=jax.ShapeDtypeStruct((NC, n_out, d), jnp.float32),
        mesh=mesh,
        scratch_types=scratch,
        compiler_params=pltpu.CompilerParams(use_tc_tiling_on_sc=False),
    )


N_PAD = 10240

_deg_call = _scatter_rows(256, N_PAD, 8, 80, 125, gather=False)
_layer64_call = _scatter_rows(N_NODES, N_PAD, 64, 80, 125, gather=True)
_pool_call = _scatter_rows(N_NODES, 128, 128, 4, 80, gather=True)


def _pad2(w, rows, cols):
    return jnp.zeros((rows, cols), w.dtype).at[:w.shape[0], :w.shape[1]].set(w)


def kernel(x, edge_index, batch, W1, b1, W2, b2, W3, b3, W4, b4,
           fcW1, fcb1, fcW2, fcb2):
    src3 = edge_index[0].reshape(NW, 80, 125)
    dst3 = edge_index[1].reshape(NW, 80, 125)

    ones_rows = jnp.zeros((256, 8), jnp.float32).at[:125].set(1.0)
    degp = _deg_call(ones_rows, src3, dst3)
    deg = degp[0, :N_NODES, 0] + degp[1, :N_NODES, 0] + 1.0
    dinv = lax.rsqrt(deg)

    def conv(h, W, b, dpad):
        t = (h @ _pad2(W, h.shape[1], dpad)) * dinv[:, None]
        s_parts = []
        for i in range(dpad // 64):
            tc = t[:, i * 64:(i + 1) * 64]
            p = _layer64_call(tc, src3, dst3)
            s_parts.append(p[0, :N_NODES] + p[1, :N_NODES] + tc)
        s = jnp.concatenate(s_parts, 1) if len(s_parts) > 1 else s_parts[0]
        bp = jnp.zeros((dpad,), b.dtype).at[:b.shape[0]].set(b)
        return jax.nn.relu(s * dinv[:, None] + bp)

    h = conv(x, W1, b1, 64)
    h = conv(h, W2, b2, 64)
    h = conv(h, W3, b3, 64)
    u = h * dinv[:, None]
    p = _layer64_call(u, src3, dst3)
    a4 = (p[0, :N_NODES] + p[1, :N_NODES] + u) * dinv[:, None]
    b4p = jnp.zeros((128,), b4.dtype).at[:b4.shape[0]].set(b4)
    h = jax.nn.relu(a4 @ _pad2(W4, 64, 128) + b4p)

    h = h.at[:, 100].set(1.0)
    npad = 240
    srcp = jnp.concatenate(
        [jnp.arange(N_NODES, dtype=jnp.int32),
         jnp.zeros((npad,), jnp.int32)]).reshape(NW, 4, 80)
    dstp = jnp.concatenate(
        [batch.astype(jnp.int32),
         jnp.full((npad,), N_GRAPHS, jnp.int32)]).reshape(NW, 4, 80)
    pparts = _pool_call(h, srcp, dstp)
    sums = (pparts[0] + pparts[1])[:N_GRAPHS]
    counts = sums[:, 100]
    pooled = sums[:, :100] / jnp.maximum(counts, 1.0)[:, None]

    z = pooled @ fcW1 + fcb1
    z = z @ fcW2 + fcb2
    return z
